# baseline (speedup 1.0000x reference)
import functools

import jax
import jax.numpy as jnp
from jax import lax
from jax.experimental import pallas as pl
from jax.experimental.pallas import tpu as pltpu
from jax.experimental.pallas import tpu_sc as plsc
from jax.experimental.layout import Layout, with_layout_constraint

_NC, _NS = 2, 16
_NW = _NC * _NS
_WINDOW = 512


def kernel(input, weight):
    batch, hist = input.shape
    _, dim = weight.shape
    n = batch * hist

    scaled = input.reshape(n) * 4
    groups = scaled.reshape(-1, 4)
    L = jnp.repeat(groups.T, 4, axis=1)

    weight = with_layout_constraint(
        weight, Layout(major_to_minor=(0, 1), tiling=((32,),))
    )

    per_w = n // _NW
    n_win = per_w // _WINDOW
    q = n // 4
    out_sds = jax.ShapeDtypeStruct((q, dim), jnp.float32)

    mesh = plsc.VectorSubcoreMesh(core_axis_name="c", subcore_axis_name="s")

    @functools.partial(
        pl.kernel,
        mesh=mesh,
        out_type=(out_sds, out_sds, out_sds, out_sds),
        scratch_types=[
            pltpu.VMEM((_WINDOW,), jnp.int32),
            pltpu.VMEM((_WINDOW, dim), jnp.float32),
            pltpu.SemaphoreType.DMA,
        ],
    )
    def gather_kernel(table_hbm, l_hbm, o0, o1, o2, o3, idx_v, rows_v, sem):
        wid = lax.axis_index("s") * _NC + lax.axis_index("c")
        outs = (o0, o1, o2, o3)

        for j in range(4):
            out_hbm = outs[j]

            @pl.loop(0, n_win)
            def _(win):
                base = (wid * n_win + win) * _WINDOW
                pltpu.sync_copy(l_hbm.at[j, pl.ds(base, _WINDOW)], idx_v)
                pltpu.async_copy(table_hbm.at[idx_v], rows_v, sem).wait()
                base4 = pl.multiple_of(base // 4, _WINDOW // 4)
                pltpu.sync_copy(
                    rows_v.at[pl.ds(0, _WINDOW // 4)],
                    out_hbm.at[pl.ds(base4, _WINDOW // 4)],
                )

    o0, o1, o2, o3 = gather_kernel(weight, L)
    out = jnp.stack([o0, o1, o2, o3], axis=1)
    return out.reshape(batch, hist, dim)
